# baseline (speedup 1.0000x reference)
import jax
import jax.numpy as jnp
from jax.experimental import pallas as pl


def _pos_kernel(col_ref, row_ref, out_ref):
    d, w = col_ref.shape
    h_blk = row_ref.shape[0]
    col = col_ref[...]
    row = jnp.transpose(row_ref[...], (1, 0))
    out_ref[0, :d] = jnp.broadcast_to(col[:, None, :], (d, h_blk, w))
    out_ref[0, d:] = jnp.broadcast_to(row[:, :, None], (d, h_blk, w))


def kernel(x, row_w, col_w):
    b = x.shape[0]
    h, w = x.shape[-2], x.shape[-1]
    d = row_w.shape[-1]
    col_t = col_w[:w].T
    row_s = row_w[:h]

    h_blk = 16
    n_h = h // h_blk

    out = pl.pallas_call(
        _pos_kernel,
        grid=(b, n_h),
        in_specs=[
            pl.BlockSpec((d, w), lambda bi, hi: (0, 0)),
            pl.BlockSpec((h_blk, d), lambda bi, hi: (hi, 0)),
        ],
        out_specs=pl.BlockSpec((1, 2 * d, h_blk, w), lambda bi, hi: (bi, 0, hi, 0)),
        out_shape=jax.ShapeDtypeStruct((b, 2 * d, h, w), jnp.float32),
    )(col_t, row_s)
    return out.reshape(b, 2 * d, h * w)
